# Initial kernel scaffold; baseline (speedup 1.0000x reference)
import jax
import jax.numpy as jnp
from jax.experimental import pallas as pl


def kernel(pitch, amplitude, wavetables, attention):
    raise NotImplementedError("write your pallas kernel here")


# speedup vs baseline: 1871.4920x; 1871.4920x over previous
import functools

import jax
import jax.numpy as jnp
from jax import lax
from jax.experimental import pallas as pl
from jax.experimental.pallas import tpu as pltpu
from jax.experimental.pallas import tpu_sc as plsc

SR = 16000
L = 512
B = 16
T = 64000
NWT = 10
CHUNK = 512

TOTAL = B * T
NW = 32
SPAN = TOTAL // NW
SUB = 16000
VEC = 16


def _tc_prep_body(pitch_ref, wt_ref, att_ref, il_ref, alpha_ref, c_ref,
                  d_ref, carry_ref):
    step = pl.program_id(0)

    @pl.when(step == 0)
    def _init():
        carry_ref[...] = jnp.zeros((B, 1), jnp.float32)
        c = wt_ref[:, 0, :] * att_ref[:, 0:1]
        for w in range(1, NWT):
            c = c + wt_ref[:, w, :] * att_ref[:, w:w + 1]
        c_ref[...] = c
        d_ref[...] = jnp.concatenate([c[:, 1:], c[:, :1]], axis=1) - c

    inc = pitch_ref[...] * (float(L) / float(SR))
    cs = inc
    k = 1
    while k < CHUNK:
        cs = cs + jnp.concatenate(
            [jnp.zeros((B, k), jnp.float32), cs[:, :CHUNK - k]], axis=1)
        k *= 2
    cs = cs + carry_ref[...]
    carry_ref[...] = cs[:, CHUNK - 1:CHUNK]

    idx = cs - inc[1:2, :]
    m = idx - float(L) * jnp.floor(idx * (1.0 / float(L)))
    ilf = jnp.floor(m)
    alpha_ref[...] = m - ilf
    il = ilf.astype(jnp.int32) & (L - 1)
    b_iota = lax.broadcasted_iota(jnp.int32, (B, CHUNK), 0)
    il_ref[...] = il + b_iota * L


_tc_prep = pl.pallas_call(
    _tc_prep_body,
    grid=(T // CHUNK,),
    in_specs=[
        pl.BlockSpec((B, CHUNK), lambda s: (0, s)),
        pl.BlockSpec((B, NWT, L), lambda s: (0, 0, 0)),
        pl.BlockSpec((B, NWT), lambda s: (0, 0)),
    ],
    out_specs=[
        pl.BlockSpec((B, CHUNK), lambda s: (0, s)),
        pl.BlockSpec((B, CHUNK), lambda s: (0, s)),
        pl.BlockSpec((B, L), lambda s: (0, 0)),
        pl.BlockSpec((B, L), lambda s: (0, 0)),
    ],
    out_shape=[
        jax.ShapeDtypeStruct((B, T), jnp.int32),
        jax.ShapeDtypeStruct((B, T), jnp.float32),
        jax.ShapeDtypeStruct((B, L), jnp.float32),
        jax.ShapeDtypeStruct((B, L), jnp.float32),
    ],
    scratch_shapes=[pltpu.VMEM((B, 1), jnp.float32)],
    compiler_params=pltpu.CompilerParams(
        dimension_semantics=("arbitrary",)),
)


@functools.cache
def _build_sc_gather():
    mesh = plsc.VectorSubcoreMesh(core_axis_name="c", subcore_axis_name="s",
                                  num_cores=2, num_subcores=16)

    @functools.partial(
        pl.kernel,
        out_type=jax.ShapeDtypeStruct((TOTAL,), jnp.float32),
        mesh=mesh,
        scratch_types=[
            pltpu.VMEM((B * L,), jnp.float32),
            pltpu.VMEM((B * L,), jnp.float32),
            pltpu.VMEM((SUB,), jnp.int32),
            pltpu.VMEM((SUB,), jnp.float32),
            pltpu.VMEM((SUB,), jnp.float32),
            pltpu.VMEM((SUB,), jnp.float32),
            pltpu.SemaphoreType.DMA,
        ],
        compiler_params=pltpu.CompilerParams(needs_layout_passes=False),
    )
    def _sc_gather(il_hbm, alpha_hbm, amp_hbm, c_hbm, d_hbm, out_hbm,
                   cv, dv, idxv, av, mv, ov, sem):
        wid = lax.axis_index("s") * 2 + lax.axis_index("c")
        base = wid * SPAN
        ctb = pltpu.async_copy(c_hbm, cv, sem)
        dtb = pltpu.async_copy(d_hbm, dv, sem)
        ctb.wait()
        dtb.wait()

        @pl.loop(0, SPAN, step=SUB)
        def _chunk(off):
            start = base + off
            c1 = pltpu.async_copy(il_hbm.at[pl.ds(start, SUB)], idxv, sem)
            c2 = pltpu.async_copy(alpha_hbm.at[pl.ds(start, SUB)], av, sem)
            c3 = pltpu.async_copy(amp_hbm.at[pl.ds(start, SUB)], mv, sem)
            c1.wait()
            c2.wait()
            c3.wait()

            @pl.loop(0, SUB, step=VEC)
            def _vec(c0):
                iv = idxv[pl.ds(c0, VEC)]
                lo = plsc.load_gather(cv, [iv])
                sl = plsc.load_gather(dv, [iv])
                a = av[pl.ds(c0, VEC)]
                amp = mv[pl.ds(c0, VEC)]
                ov[pl.ds(c0, VEC)] = amp * (lo + a * sl)

            pltpu.sync_copy(ov, out_hbm.at[pl.ds(start, SUB)])

    return _sc_gather


def kernel(pitch, amplitude, wavetables, attention):
    il, alpha, c, d = _tc_prep(pitch, wavetables, attention)
    sc_gather = _build_sc_gather()
    out = sc_gather(il.reshape(-1), alpha.reshape(-1),
                    amplitude.reshape(-1), c.reshape(-1), d.reshape(-1))
    return out.reshape(B, T, 1), attention


# speedup vs baseline: 3972.7826x; 2.1228x over previous
import functools

import jax
import jax.numpy as jnp
from jax import lax
from jax.experimental import pallas as pl
from jax.experimental.pallas import tpu as pltpu
from jax.experimental.pallas import tpu_sc as plsc

SR = 16000
L = 512
B = 16
T = 64000
NWT = 10
CHUNK = 16000

TOTAL = B * T
NW = 32
SPAN = TOTAL // NW
SUB = 16000
VEC = 16


def _tc_prep_body(pitch_ref, wt_ref, att_ref, il_ref, alpha_ref, c_ref,
                  d_ref, carry_ref):
    step = pl.program_id(0)

    @pl.when(step == 0)
    def _init():
        carry_ref[...] = jnp.zeros((B, 1), jnp.float32)
        c = wt_ref[:, 0, :] * att_ref[:, 0:1]
        for w in range(1, NWT):
            c = c + wt_ref[:, w, :] * att_ref[:, w:w + 1]
        c_ref[...] = c
        d_ref[...] = jnp.concatenate([c[:, 1:], c[:, :1]], axis=1) - c

    inc = pitch_ref[...] * (float(L) / float(SR))
    cs = inc
    k = 1
    while k < CHUNK:
        cs = cs + jnp.concatenate(
            [jnp.zeros((B, k), jnp.float32), cs[:, :CHUNK - k]], axis=1)
        k *= 2
    cs = cs + carry_ref[...]
    carry_ref[...] = cs[:, CHUNK - 1:CHUNK]

    idx = cs - inc[1:2, :]
    m = idx - float(L) * jnp.floor(idx * (1.0 / float(L)))
    ilf = jnp.floor(m)
    alpha_ref[...] = m - ilf
    il = ilf.astype(jnp.int32) & (L - 1)
    b_iota = lax.broadcasted_iota(jnp.int32, (B, CHUNK), 0)
    il_ref[...] = il + b_iota * L


_tc_prep = pl.pallas_call(
    _tc_prep_body,
    grid=(T // CHUNK,),
    in_specs=[
        pl.BlockSpec((B, CHUNK), lambda s: (0, s)),
        pl.BlockSpec((B, NWT, L), lambda s: (0, 0, 0)),
        pl.BlockSpec((B, NWT), lambda s: (0, 0)),
    ],
    out_specs=[
        pl.BlockSpec((B, CHUNK), lambda s: (0, s)),
        pl.BlockSpec((B, CHUNK), lambda s: (0, s)),
        pl.BlockSpec((B, L), lambda s: (0, 0)),
        pl.BlockSpec((B, L), lambda s: (0, 0)),
    ],
    out_shape=[
        jax.ShapeDtypeStruct((B, T), jnp.int32),
        jax.ShapeDtypeStruct((B, T), jnp.float32),
        jax.ShapeDtypeStruct((B, L), jnp.float32),
        jax.ShapeDtypeStruct((B, L), jnp.float32),
    ],
    scratch_shapes=[pltpu.VMEM((B, 1), jnp.float32)],
    compiler_params=pltpu.CompilerParams(
        dimension_semantics=("arbitrary",)),
)


@functools.cache
def _build_sc_gather():
    mesh = plsc.VectorSubcoreMesh(core_axis_name="c", subcore_axis_name="s",
                                  num_cores=2, num_subcores=16)

    @functools.partial(
        pl.kernel,
        out_type=jax.ShapeDtypeStruct((TOTAL,), jnp.float32),
        mesh=mesh,
        scratch_types=[
            pltpu.VMEM((B * L,), jnp.float32),
            pltpu.VMEM((B * L,), jnp.float32),
            pltpu.VMEM((SUB,), jnp.int32),
            pltpu.VMEM((SUB,), jnp.float32),
            pltpu.VMEM((SUB,), jnp.float32),
            pltpu.VMEM((SUB,), jnp.float32),
            pltpu.SemaphoreType.DMA,
        ],
        compiler_params=pltpu.CompilerParams(needs_layout_passes=False),
    )
    def _sc_gather(il_hbm, alpha_hbm, amp_hbm, c_hbm, d_hbm, out_hbm,
                   cv, dv, idxv, av, mv, ov, sem):
        wid = lax.axis_index("s") * 2 + lax.axis_index("c")
        base = wid * SPAN
        ctb = pltpu.async_copy(c_hbm, cv, sem)
        dtb = pltpu.async_copy(d_hbm, dv, sem)
        ctb.wait()
        dtb.wait()

        @pl.loop(0, SPAN, step=SUB)
        def _chunk(off):
            start = base + off
            c1 = pltpu.async_copy(il_hbm.at[pl.ds(start, SUB)], idxv, sem)
            c2 = pltpu.async_copy(alpha_hbm.at[pl.ds(start, SUB)], av, sem)
            c3 = pltpu.async_copy(amp_hbm.at[pl.ds(start, SUB)], mv, sem)
            c1.wait()
            c2.wait()
            c3.wait()

            @pl.loop(0, SUB, step=VEC)
            def _vec(c0):
                iv = idxv[pl.ds(c0, VEC)]
                lo = plsc.load_gather(cv, [iv])
                sl = plsc.load_gather(dv, [iv])
                a = av[pl.ds(c0, VEC)]
                amp = mv[pl.ds(c0, VEC)]
                ov[pl.ds(c0, VEC)] = amp * (lo + a * sl)

            pltpu.sync_copy(ov, out_hbm.at[pl.ds(start, SUB)])

    return _sc_gather


def kernel(pitch, amplitude, wavetables, attention):
    il, alpha, c, d = _tc_prep(pitch, wavetables, attention)
    sc_gather = _build_sc_gather()
    out = sc_gather(il.reshape(-1), alpha.reshape(-1),
                    amplitude.reshape(-1), c.reshape(-1), d.reshape(-1))
    return out.reshape(B, T, 1), attention


# speedup vs baseline: 5558.9329x; 1.3993x over previous
import functools

import jax
import jax.numpy as jnp
from jax import lax
from jax.experimental import pallas as pl
from jax.experimental.pallas import tpu as pltpu
from jax.experimental.pallas import tpu_sc as plsc

SR = 16000
L = 512
B = 16
T = 64000
NWT = 10
CHUNK = 16000

TOTAL = B * T
NW = 32
SPAN = TOTAL // NW
SUB = 8000
NSUB = SPAN // SUB
VEC = 16
MAXFRAC = float(L) - 2.0 ** -11


def _tc_prep_body(pitch_ref, wt_ref, att_ref, packed_ref, c_ref,
                  d_ref, carry_ref):
    step = pl.program_id(0)

    @pl.when(step == 0)
    def _init():
        carry_ref[...] = jnp.zeros((B, 1), jnp.float32)
        c = wt_ref[:, 0, :] * att_ref[:, 0:1]
        for w in range(1, NWT):
            c = c + wt_ref[:, w, :] * att_ref[:, w:w + 1]
        c_ref[...] = c
        d_ref[...] = jnp.concatenate([c[:, 1:], c[:, :1]], axis=1) - c

    inc = pitch_ref[...] * (float(L) / float(SR))
    cs = inc
    k = 1
    while k < CHUNK:
        cs = cs + jnp.concatenate(
            [jnp.zeros((B, k), jnp.float32), cs[:, :CHUNK - k]], axis=1)
        k *= 2
    cs = cs + carry_ref[...]
    carry_ref[...] = cs[:, CHUNK - 1:CHUNK]

    idx = cs - inc[1:2, :]
    m = idx - float(L) * jnp.floor(idx * (1.0 / float(L)))
    rowbase = (lax.broadcasted_iota(jnp.int32, (B, CHUNK), 0) * L
               ).astype(jnp.float32)
    packed_ref[...] = jnp.minimum(rowbase + m, rowbase + MAXFRAC)


_tc_prep = pl.pallas_call(
    _tc_prep_body,
    grid=(T // CHUNK,),
    in_specs=[
        pl.BlockSpec((B, CHUNK), lambda s: (0, s)),
        pl.BlockSpec((B, NWT, L), lambda s: (0, 0, 0)),
        pl.BlockSpec((B, NWT), lambda s: (0, 0)),
    ],
    out_specs=[
        pl.BlockSpec((B, CHUNK), lambda s: (0, s)),
        pl.BlockSpec((B, L), lambda s: (0, 0)),
        pl.BlockSpec((B, L), lambda s: (0, 0)),
    ],
    out_shape=[
        jax.ShapeDtypeStruct((B, T), jnp.float32),
        jax.ShapeDtypeStruct((B, L), jnp.float32),
        jax.ShapeDtypeStruct((B, L), jnp.float32),
    ],
    scratch_shapes=[pltpu.VMEM((B, 1), jnp.float32)],
    compiler_params=pltpu.CompilerParams(
        dimension_semantics=("arbitrary",)),
)


@functools.cache
def _build_sc_gather():
    mesh = plsc.VectorSubcoreMesh(core_axis_name="c", subcore_axis_name="s",
                                  num_cores=2, num_subcores=16)

    @functools.partial(
        pl.kernel,
        out_type=jax.ShapeDtypeStruct((TOTAL,), jnp.float32),
        mesh=mesh,
        scratch_types=[
            pltpu.VMEM((B * L,), jnp.float32),
            pltpu.VMEM((B * L,), jnp.float32),
            pltpu.VMEM((SUB,), jnp.float32),
            pltpu.VMEM((SUB,), jnp.float32),
            pltpu.VMEM((SUB,), jnp.float32),
            pltpu.VMEM((SUB,), jnp.float32),
            pltpu.VMEM((SUB,), jnp.float32),
            pltpu.VMEM((SUB,), jnp.float32),
            pltpu.SemaphoreType.DMA,
            pltpu.SemaphoreType.DMA,
            pltpu.SemaphoreType.DMA,
            pltpu.SemaphoreType.DMA,
            pltpu.SemaphoreType.DMA,
        ],
        compiler_params=pltpu.CompilerParams(needs_layout_passes=False),
    )
    def _sc_gather(packed_hbm, amp_hbm, c_hbm, d_hbm, out_hbm,
                   cv, dv, pv0, pv1, mv0, mv1, ov0, ov1,
                   tsem, isem0, isem1, osem0, osem1):
        wid = lax.axis_index("s") * 2 + lax.axis_index("c")
        base = wid * SPAN
        pvs, mvs, ovs = (pv0, pv1), (mv0, mv1), (ov0, ov1)
        isems = (isem0, isem1)
        osems = (osem0, osem1)
        ctb = pltpu.async_copy(c_hbm, cv, tsem)
        dtb = pltpu.async_copy(d_hbm, dv, tsem)

        def start_in(g, slot):
            start = base + g * SUB
            return (
                pltpu.async_copy(packed_hbm.at[pl.ds(start, SUB)],
                                 pvs[slot], isems[slot]),
                pltpu.async_copy(amp_hbm.at[pl.ds(start, SUB)],
                                 mvs[slot], isems[slot]),
            )

        pending = {0: start_in(0, 0)}
        outcopies = {}
        ctb.wait()
        dtb.wait()
        for g in range(NSUB):
            slot = g & 1
            if g + 1 < NSUB:
                pending[g + 1] = start_in(g + 1, (g + 1) & 1)
            for cp in pending.pop(g):
                cp.wait()
            if g >= 2:
                outcopies.pop(g - 2).wait()
            pslot, mslot, oslot = pvs[slot], mvs[slot], ovs[slot]

            @plsc.parallel_loop(0, SUB, step=VEC, unroll=8)
            def _vec(c0):
                p = pslot[pl.ds(c0, VEC)]
                iv = p.astype(jnp.int32)
                a = p - iv.astype(jnp.float32)
                lo = plsc.load_gather(cv, [iv])
                sl = plsc.load_gather(dv, [iv])
                amp = mslot[pl.ds(c0, VEC)]
                oslot[pl.ds(c0, VEC)] = amp * (lo + a * sl)

            outcopies[g] = pltpu.async_copy(
                oslot, out_hbm.at[pl.ds(base + g * SUB, SUB)], osems[slot])
        for g in sorted(outcopies):
            outcopies[g].wait()

    return _sc_gather


def kernel(pitch, amplitude, wavetables, attention):
    packed, c, d = _tc_prep(pitch, wavetables, attention)
    sc_gather = _build_sc_gather()
    out = sc_gather(packed.reshape(-1), amplitude.reshape(-1),
                    c.reshape(-1), d.reshape(-1))
    return out.reshape(B, T, 1), attention


# speedup vs baseline: 6288.8889x; 1.1313x over previous
import functools

import jax
import jax.numpy as jnp
from jax import lax
from jax.experimental import pallas as pl
from jax.experimental.pallas import tpu as pltpu
from jax.experimental.pallas import tpu_sc as plsc

SR = 16000
L = 512
B = 16
T = 64000
NWT = 10
CHUNK = 16000

TOTAL = B * T
NW = 32
SPAN = TOTAL // NW
SUB = 16000
NSUB = SPAN // SUB
VEC = 16
MAXFRAC = float(L) - 2.0 ** -11


def _tc_prep_body(pitch_ref, wt_ref, att_ref, packed_ref, e_ref, carry_ref):
    step = pl.program_id(0)

    @pl.when(step == 0)
    def _init():
        carry_ref[...] = jnp.zeros((B, 1), jnp.float32)
        c = wt_ref[:, 0, :] * att_ref[:, 0:1]
        for w in range(1, NWT):
            c = c + wt_ref[:, w, :] * att_ref[:, w:w + 1]
        d = jnp.concatenate([c[:, 1:], c[:, :1]], axis=1) - c
        cbits = lax.bitcast_convert_type(
            c.astype(jnp.bfloat16).astype(jnp.float32), jnp.int32)
        dbits = lax.bitcast_convert_type(
            d.astype(jnp.bfloat16).astype(jnp.float32), jnp.int32)
        e_ref[...] = (lax.shift_right_logical(cbits, 16)
                      | (dbits & jnp.int32(-65536)))

    inc = pitch_ref[...] * (float(L) / float(SR))
    cs = inc
    k = 1
    while k < CHUNK:
        cs = cs + jnp.concatenate(
            [jnp.zeros((B, k), jnp.float32), cs[:, :CHUNK - k]], axis=1)
        k *= 2
    cs = cs + carry_ref[...]
    carry_ref[...] = cs[:, CHUNK - 1:CHUNK]

    idx = cs - inc[1:2, :]
    m = jnp.maximum(idx - float(L) * jnp.floor(idx * (1.0 / float(L))), 0.0)
    rowbase = (lax.broadcasted_iota(jnp.int32, (B, CHUNK), 0) * L
               ).astype(jnp.float32)
    packed_ref[...] = jnp.minimum(rowbase + m, rowbase + MAXFRAC)


_tc_prep = pl.pallas_call(
    _tc_prep_body,
    grid=(T // CHUNK,),
    in_specs=[
        pl.BlockSpec((B, CHUNK), lambda s: (0, s)),
        pl.BlockSpec((B, NWT, L), lambda s: (0, 0, 0)),
        pl.BlockSpec((B, NWT), lambda s: (0, 0)),
    ],
    out_specs=[
        pl.BlockSpec((B, CHUNK), lambda s: (0, s)),
        pl.BlockSpec((B, L), lambda s: (0, 0)),
    ],
    out_shape=[
        jax.ShapeDtypeStruct((B, T), jnp.float32),
        jax.ShapeDtypeStruct((B, L), jnp.int32),
    ],
    scratch_shapes=[pltpu.VMEM((B, 1), jnp.float32)],
    compiler_params=pltpu.CompilerParams(
        dimension_semantics=("arbitrary",)),
)


@functools.cache
def _build_sc_gather():
    mesh = plsc.VectorSubcoreMesh(core_axis_name="c", subcore_axis_name="s",
                                  num_cores=2, num_subcores=16)

    @functools.partial(
        pl.kernel,
        out_type=jax.ShapeDtypeStruct((TOTAL,), jnp.float32),
        mesh=mesh,
        scratch_types=[
            pltpu.VMEM((B * L,), jnp.int32),
            pltpu.VMEM((SUB,), jnp.float32),
            pltpu.VMEM((SUB,), jnp.float32),
            pltpu.VMEM((SUB,), jnp.float32),
            pltpu.VMEM((SUB,), jnp.float32),
            pltpu.VMEM((SUB,), jnp.float32),
            pltpu.VMEM((SUB,), jnp.float32),
            pltpu.SemaphoreType.DMA,
            pltpu.SemaphoreType.DMA,
            pltpu.SemaphoreType.DMA,
            pltpu.SemaphoreType.DMA,
            pltpu.SemaphoreType.DMA,
        ],
        compiler_params=pltpu.CompilerParams(needs_layout_passes=False,
                                             use_tc_tiling_on_sc=True),
    )
    def _sc_gather(packed_hbm, amp_hbm, e_hbm, out_hbm,
                   ev, pv0, pv1, mv0, mv1, ov0, ov1,
                   tsem, isem0, isem1, osem0, osem1):
        wid = lax.axis_index("s") * 2 + lax.axis_index("c")
        row = wid // 2
        col0 = (wid % 2) * SPAN
        base = row * T + col0
        pvs, mvs, ovs = (pv0, pv1), (mv0, mv1), (ov0, ov1)
        isems = (isem0, isem1)
        osems = (osem0, osem1)
        etb = pltpu.async_copy(e_hbm, ev, tsem)

        def start_in(g, slot):
            start = base + g * SUB
            return (
                pltpu.async_copy(
                    packed_hbm.at[row, pl.ds(col0 + g * SUB, SUB)],
                    pvs[slot], isems[slot]),
                pltpu.async_copy(amp_hbm.at[pl.ds(start, SUB)],
                                 mvs[slot], isems[slot]),
            )

        pending = {0: start_in(0, 0)}
        outcopies = {}
        etb.wait()
        for g in range(NSUB):
            slot = g & 1
            if g + 1 < NSUB:
                pending[g + 1] = start_in(g + 1, (g + 1) & 1)
            for cp in pending.pop(g):
                cp.wait()
            if g >= 2:
                outcopies.pop(g - 2).wait()
            pslot, mslot, oslot = pvs[slot], mvs[slot], ovs[slot]

            @plsc.parallel_loop(0, SUB, step=VEC, unroll=8)
            def _vec(c0):
                p = pslot[pl.ds(c0, VEC)]
                iv = p.astype(jnp.int32)
                a = p - iv.astype(jnp.float32)
                cd = plsc.load_gather(ev, [iv])
                lo = plsc.bitcast(lax.shift_left(cd, 16), jnp.float32)
                sl = plsc.bitcast(cd & jnp.int32(-65536), jnp.float32)
                amp = mslot[pl.ds(c0, VEC)]
                oslot[pl.ds(c0, VEC)] = amp * (lo + a * sl)

            outcopies[g] = pltpu.async_copy(
                oslot, out_hbm.at[pl.ds(base + g * SUB, SUB)], osems[slot])
        for g in sorted(outcopies):
            outcopies[g].wait()

    return _sc_gather


def kernel(pitch, amplitude, wavetables, attention):
    packed, e = _tc_prep(pitch, wavetables, attention)
    sc_gather = _build_sc_gather()
    out = sc_gather(packed, amplitude.reshape(-1), e.reshape(-1))
    return out.reshape(B, T, 1), attention
